# baseline (speedup 1.0000x reference)
import functools

import jax
import jax.numpy as jnp
from jax import lax
from jax.experimental import pallas as pl
from jax.experimental.pallas import tpu as pltpu
from jax.experimental.pallas import tpu_sc as plsc

B, D, T = 16, 256, 576
K = 8192
N = B * T
TK = 512
NCHUNK = K // TK

NC, NS, L = 2, 16, 16
NW = NC * NS
TOK_W = N // NW
IDX_COLS = 96
IDX_ROWS_W = (N // IDX_COLS) // NW
CODES_W = K // NW


def _tc_dist_argmin(x_ref, e_ref, idx_ref, loss_ref):
    b = pl.program_id(0)
    xb = x_ref[0]
    x2 = jnp.sum(xb * xb, axis=0)
    best = jnp.full((T,), jnp.inf, jnp.float32)
    besti = jnp.zeros((T,), jnp.int32)
    for kk in range(NCHUNK):
        e = e_ref[pl.ds(kk * TK, TK), :]
        e2 = jnp.sum(e * e, axis=1)
        mm = lax.dot_general(e, xb, (((1,), (0,)), ((), ())),
                             preferred_element_type=jnp.float32)
        d2 = (x2[None, :] + e2[:, None]) - 2.0 * mm
        dist = jnp.sqrt(jnp.maximum(d2, 0.0))
        m = jnp.min(dist, axis=0)
        row = lax.broadcasted_iota(jnp.int32, (TK, T), 0)
        a = jnp.min(jnp.where(dist == m[None, :], row, K), axis=0) + kk * TK
        upd = m < best
        besti = jnp.where(upd, a, besti)
        best = jnp.where(upd, m, best)
    idx_ref[0, 0, :] = besti
    part = jnp.sum(best * best)

    @pl.when(b == 0)
    def _():
        loss_ref[...] = jnp.zeros_like(loss_ref)

    loss_ref[...] = loss_ref[...] + part

    @pl.when(b == B - 1)
    def _():
        loss_ref[...] = loss_ref[...] * (1.0 / (N * D))


_tc_call = pl.pallas_call(
    _tc_dist_argmin,
    grid=(B,),
    in_specs=[
        pl.BlockSpec((1, D, T), lambda b: (b, 0, 0)),
        pl.BlockSpec((K, D), lambda b: (0, 0)),
    ],
    out_specs=[
        pl.BlockSpec((1, 1, T), lambda b: (b, 0, 0)),
        pl.BlockSpec((1, 1), lambda b: (0, 0)),
    ],
    out_shape=[
        jax.ShapeDtypeStruct((B, 1, T), jnp.int32),
        jax.ShapeDtypeStruct((1, 1), jnp.float32),
    ],
)


def _sc_body(embed_hbm, idx_hbm, idxb_hbm, q_hbm, part_hbm,
             idx_v, rows_v, all_v, flags_v, stage_v, sem):
    w = lax.axis_index("s") * NC + lax.axis_index("c")

    pltpu.sync_copy(idx_hbm.at[w], idx_v)
    handles = []
    for j in range(IDX_ROWS_W):
        handles.append(pltpu.async_copy(
            embed_hbm.at[idx_v.at[j]],
            rows_v.at[pl.ds(j * IDX_COLS, IDX_COLS)], sem))
    for h in handles:
        h.wait()
    pltpu.sync_copy(rows_v, q_hbm.at[pl.ds(w * TOK_W, TOK_W)])

    pltpu.sync_copy(idxb_hbm, all_v)
    zeros16 = jnp.zeros((L,), jnp.float32)
    ones16 = jnp.ones((L,), jnp.float32)
    for c in range(CODES_W // L):
        flags_v[pl.ds(c * L, L)] = zeros16
    lo = w * CODES_W

    def row_body(r, carry):
        for c in range(128 // L):
            v = all_v[r, pl.ds(c * L, L)]
            msk = (v >= lo) & (v < lo + CODES_W)
            off = jnp.clip(v - lo, 0, CODES_W - 1)
            plsc.store_scatter(flags_v, [off], ones16, mask=msk)
        return carry

    lax.fori_loop(0, N // 128, row_body, 0)

    acc = jnp.zeros((L,), jnp.float32)
    for c in range(CODES_W // L):
        acc = acc + flags_v[pl.ds(c * L, L)]
    cnt = jnp.sum(acc)
    lane = lax.iota(jnp.int32, L)
    stage_v[...] = jnp.where(lane == 0, cnt, 0.0)
    pltpu.sync_copy(stage_v, part_hbm.at[pl.ds(w * L, L)])


@functools.lru_cache(maxsize=1)
def _get_sc_call():
    return pl.kernel(
        _sc_body,
        out_type=[
            jax.ShapeDtypeStruct((N, D), jnp.float32),
            jax.ShapeDtypeStruct((NW * L,), jnp.float32),
        ],
        mesh=plsc.VectorSubcoreMesh(core_axis_name="c", subcore_axis_name="s",
                                    num_cores=NC, num_subcores=NS),
        compiler_params=pltpu.CompilerParams(needs_layout_passes=False),
        scratch_types=[
            pltpu.VMEM((IDX_ROWS_W, IDX_COLS), jnp.int32),
            pltpu.VMEM((TOK_W, D), jnp.float32),
            pltpu.VMEM((N // 128, 128), jnp.int32),
            pltpu.VMEM((CODES_W,), jnp.float32),
            pltpu.VMEM((L,), jnp.float32),
            pltpu.SemaphoreType.DMA,
        ],
    )


def kernel(x, embed):
    idx16, loss = _tc_call(x, embed)
    idx_flat = idx16.reshape(N)
    idx2 = idx_flat.reshape(NW, IDX_ROWS_W, IDX_COLS)
    idxb = idx_flat.reshape(N // 128, 128)
    q, part = _get_sc_call()(embed, idx2, idxb)
    quantized_st = jnp.transpose(q.reshape(B, T, D), (0, 2, 1))
    util = jnp.sum(part) * (1.0 / K)
    return quantized_st, idx_flat, loss[0, 0], util


# speedup vs baseline: 1.2066x; 1.2066x over previous
import functools

import jax
import jax.numpy as jnp
from jax import lax
from jax.experimental import pallas as pl
from jax.experimental.pallas import tpu as pltpu
from jax.experimental.pallas import tpu_sc as plsc

B, D, T = 16, 256, 576
K = 8192
N = B * T
TK = 512
NCHUNK = K // TK

NC, NS, L = 2, 16, 16
NW = NC * NS
TOK_W = N // NW
IDX_COLS = 96
IDX_ROWS_W = (N // IDX_COLS) // NW
CODES_W = K // NW


def _nextf(t):
    return lax.bitcast_convert_type(
        lax.bitcast_convert_type(t, jnp.int32) + 1, jnp.float32)


def _prevf(t):
    return jnp.where(
        t > 0.0,
        lax.bitcast_convert_type(
            lax.bitcast_convert_type(t, jnp.int32) - 1, jnp.float32),
        0.0)


def _tc_dist_argmin(x_ref, e_ref, idx_ref, loss_ref, scr_ref):
    b = pl.program_id(0)
    xb = x_ref[0]
    x2 = jnp.sum(xb * xb, axis=0)
    m = jnp.full((T,), jnp.inf, jnp.float32)
    for kk in range(NCHUNK):
        e = e_ref[pl.ds(kk * TK, TK), :]
        e2 = jnp.sum(e * e, axis=1)
        mm = lax.dot_general(e, xb, (((1,), (0,)), ((), ())),
                             preferred_element_type=jnp.float32)
        d2 = jnp.maximum((x2[None, :] + e2[:, None]) - 2.0 * mm, 0.0)
        scr_ref[pl.ds(kk * TK, TK), :] = d2
        m = jnp.minimum(m, jnp.min(d2, axis=0))
    s = jnp.sqrt(m)
    t = s * _nextf(s)
    for _ in range(3):
        t = jnp.where(jnp.sqrt(t) > s, _prevf(t), t)
    for _ in range(3):
        tn = _nextf(t)
        t = jnp.where(jnp.sqrt(tn) <= s, tn, t)
    hi = t[None, :]
    besti = jnp.full((T,), K, jnp.int32)
    for kk in range(NCHUNK):
        d2 = scr_ref[pl.ds(kk * TK, TK), :]
        row = lax.broadcasted_iota(jnp.int32, (TK, T), 0) + kk * TK
        cand = jnp.min(jnp.where(d2 <= hi, row, K), axis=0)
        besti = jnp.minimum(besti, cand)
    idx_ref[0, 0, :] = besti
    part = jnp.sum(m)

    @pl.when(b == 0)
    def _():
        loss_ref[...] = jnp.zeros_like(loss_ref)

    loss_ref[...] = loss_ref[...] + part

    @pl.when(b == B - 1)
    def _():
        loss_ref[...] = loss_ref[...] * (1.0 / (N * D))


_tc_call = pl.pallas_call(
    _tc_dist_argmin,
    grid=(B,),
    in_specs=[
        pl.BlockSpec((1, D, T), lambda b: (b, 0, 0)),
        pl.BlockSpec((K, D), lambda b: (0, 0)),
    ],
    out_specs=[
        pl.BlockSpec((1, 1, T), lambda b: (b, 0, 0)),
        pl.BlockSpec((1, 1), lambda b: (0, 0)),
    ],
    out_shape=[
        jax.ShapeDtypeStruct((B, 1, T), jnp.int32),
        jax.ShapeDtypeStruct((1, 1), jnp.float32),
    ],
    scratch_shapes=[pltpu.VMEM((K, T), jnp.float32)],
)


def _sc_body(embed_hbm, idx_hbm, idxb_hbm, q_hbm, part_hbm,
             idx_v, rows_v, all_v, flags_v, stage_v, sem):
    w = lax.axis_index("s") * NC + lax.axis_index("c")

    pltpu.sync_copy(idx_hbm.at[w], idx_v)
    handles = []
    for j in range(IDX_ROWS_W):
        handles.append(pltpu.async_copy(
            embed_hbm.at[idx_v.at[j]],
            rows_v.at[pl.ds(j * IDX_COLS, IDX_COLS)], sem))
    for h in handles:
        h.wait()
    pltpu.sync_copy(rows_v, q_hbm.at[pl.ds(w * TOK_W, TOK_W)])

    pltpu.sync_copy(idxb_hbm, all_v)
    zeros16 = jnp.zeros((L,), jnp.float32)
    ones16 = jnp.ones((L,), jnp.float32)
    for c in range(CODES_W // L):
        flags_v[pl.ds(c * L, L)] = zeros16
    lo = w * CODES_W

    def row_body(r, carry):
        for c in range(128 // L):
            v = all_v[r, pl.ds(c * L, L)]
            msk = (v >= lo) & (v < lo + CODES_W)
            off = jnp.clip(v - lo, 0, CODES_W - 1)
            plsc.store_scatter(flags_v, [off], ones16, mask=msk)
        return carry

    lax.fori_loop(0, N // 128, row_body, 0)

    acc = jnp.zeros((L,), jnp.float32)
    for c in range(CODES_W // L):
        acc = acc + flags_v[pl.ds(c * L, L)]
    cnt = jnp.sum(acc)
    lane = lax.iota(jnp.int32, L)
    stage_v[...] = jnp.where(lane == 0, cnt, 0.0)
    pltpu.sync_copy(stage_v, part_hbm.at[pl.ds(w * L, L)])


@functools.lru_cache(maxsize=1)
def _get_sc_call():
    return pl.kernel(
        _sc_body,
        out_type=[
            jax.ShapeDtypeStruct((N, D), jnp.float32),
            jax.ShapeDtypeStruct((NW * L,), jnp.float32),
        ],
        mesh=plsc.VectorSubcoreMesh(core_axis_name="c", subcore_axis_name="s",
                                    num_cores=NC, num_subcores=NS),
        compiler_params=pltpu.CompilerParams(needs_layout_passes=False),
        scratch_types=[
            pltpu.VMEM((IDX_ROWS_W, IDX_COLS), jnp.int32),
            pltpu.VMEM((TOK_W, D), jnp.float32),
            pltpu.VMEM((N // 128, 128), jnp.int32),
            pltpu.VMEM((CODES_W,), jnp.float32),
            pltpu.VMEM((L,), jnp.float32),
            pltpu.SemaphoreType.DMA,
        ],
    )


def kernel(x, embed):
    idx16, loss = _tc_call(x, embed)
    idx_flat = idx16.reshape(N)
    idx2 = idx_flat.reshape(NW, IDX_ROWS_W, IDX_COLS)
    idxb = idx_flat.reshape(N // 128, 128)
    q, part = _get_sc_call()(embed, idx2, idxb)
    quantized_st = jnp.transpose(q.reshape(B, T, D), (0, 2, 1))
    util = jnp.sum(part) * (1.0 / K)
    return quantized_st, idx_flat, loss[0, 0], util
